# baseline (speedup 1.0000x reference)
import jax
import jax.numpy as jnp
from jax.experimental import pallas as pl


def _ma_body(map_t, actor_t, map_pe, actor_pe, time_pe, pos,
             map_o, actor_o):
    S = map_t.shape[1]
    T = actor_t.shape[1]
    N = actor_t.shape[2]
    D = map_t.shape[-1]
    pos_all = pos[...]
    map_o[...] = map_t[...] + (map_pe[...] + pos_all[:S])[None]
    time_comb = (time_pe[:T] + pos_all[:T]).reshape(1, T, 1, D)
    actor_comb = (actor_pe[:N] + pos_all[:N]).reshape(1, 1, N, D)
    actor_o[...] = actor_t[...] + actor_comb + time_comb


def _light_body(light_t, light_pe, time_pe, pos, light_o):
    T = light_t.shape[1]
    L = light_t.shape[2]
    D = light_t.shape[-1]
    pos_all = pos[...]
    time_comb = (time_pe[:T] + pos_all[:T]).reshape(1, T, 1, D)
    light_comb = (light_pe[:L] + pos_all[:L]).reshape(1, 1, L, D)
    light_o[...] = light_t[...] + light_comb + time_comb


def kernel(map_token, actor_token, light_token, map_pe_w, actor_pe_w,
           light_pe_w, time_pe_w, pos_enc):
    B, S, D = map_token.shape
    _, T, N, _ = actor_token.shape
    L = light_token.shape[2]
    G = 2 if B % 2 == 0 else 1
    whole = lambda shape: pl.BlockSpec(shape, lambda b: (0,) * len(shape))

    map_o, actor_o = pl.pallas_call(
        _ma_body,
        grid=(B // G,),
        in_specs=[
            pl.BlockSpec((G, S, D), lambda b: (b, 0, 0)),
            pl.BlockSpec((G, T, N, D), lambda b: (b, 0, 0, 0)),
            whole(map_pe_w.shape),
            whole(actor_pe_w.shape),
            whole(time_pe_w.shape),
            whole(pos_enc.shape),
        ],
        out_specs=[
            pl.BlockSpec((G, S, D), lambda b: (b, 0, 0)),
            pl.BlockSpec((G, T, N, D), lambda b: (b, 0, 0, 0)),
        ],
        out_shape=[
            jax.ShapeDtypeStruct((B, S, D), map_token.dtype),
            jax.ShapeDtypeStruct((B, T, N, D), actor_token.dtype),
        ],
    )(map_token, actor_token, map_pe_w, actor_pe_w, time_pe_w, pos_enc)

    light_o = pl.pallas_call(
        _light_body,
        grid=(B // G,),
        in_specs=[
            pl.BlockSpec((G, T, L, D), lambda b: (b, 0, 0, 0)),
            whole(light_pe_w.shape),
            whole(time_pe_w.shape),
            whole(pos_enc.shape),
        ],
        out_specs=pl.BlockSpec((G, T, L, D), lambda b: (b, 0, 0, 0)),
        out_shape=jax.ShapeDtypeStruct((B, T, L, D), light_token.dtype),
    )(light_token, light_pe_w, time_pe_w, pos_enc)

    return (map_o, actor_o, light_o)


# speedup vs baseline: 1.0538x; 1.0538x over previous
import jax
import jax.numpy as jnp
from jax.experimental import pallas as pl


def _qpe_body(map_t, actor_t, light_t, map_pe, actor_pe, light_pe, time_pe,
              pos, map_o, actor_o, light_o):
    S = map_t.shape[1]
    T = actor_t.shape[1]
    N = actor_t.shape[2]
    L = light_t.shape[2]
    D = map_t.shape[-1]

    pos_all = pos[...]
    map_o[...] = map_t[...] + (map_pe[...] + pos_all[:S])[None]

    time_comb = (time_pe[:T] + pos_all[:T]).reshape(1, T, 1, D)
    actor_comb = (actor_pe[:N] + pos_all[:N]).reshape(1, 1, N, D)
    actor_o[...] = actor_t[...] + actor_comb + time_comb

    light_comb = (light_pe[:L] + pos_all[:L]).reshape(1, 1, L, D)
    light_o[...] = light_t[...] + light_comb + time_comb


def kernel(map_token, actor_token, light_token, map_pe_w, actor_pe_w,
           light_pe_w, time_pe_w, pos_enc):
    B, S, D = map_token.shape
    _, T, N, _ = actor_token.shape
    L = light_token.shape[2]

    whole = lambda shape: pl.BlockSpec(shape, lambda b: (0,) * len(shape))
    G = 2 if B % 2 == 0 else 1
    outs = pl.pallas_call(
        _qpe_body,
        grid=(B // G,),
        in_specs=[
            pl.BlockSpec((G, S, D), lambda b: (b, 0, 0)),
            pl.BlockSpec((G, T, N, D), lambda b: (b, 0, 0, 0)),
            pl.BlockSpec((G, T, L, D), lambda b: (b, 0, 0, 0)),
            whole(map_pe_w.shape),
            whole(actor_pe_w.shape),
            whole(light_pe_w.shape),
            whole(time_pe_w.shape),
            whole(pos_enc.shape),
        ],
        out_specs=[
            pl.BlockSpec((G, S, D), lambda b: (b, 0, 0)),
            pl.BlockSpec((G, T, N, D), lambda b: (b, 0, 0, 0)),
            pl.BlockSpec((G, T, L, D), lambda b: (b, 0, 0, 0)),
        ],
        out_shape=[
            jax.ShapeDtypeStruct((B, S, D), map_token.dtype),
            jax.ShapeDtypeStruct((B, T, N, D), actor_token.dtype),
            jax.ShapeDtypeStruct((B, T, L, D), light_token.dtype),
        ],
    )(map_token, actor_token, light_token, map_pe_w, actor_pe_w,
      light_pe_w, time_pe_w, pos_enc)
    return tuple(outs)
